# baseline (speedup 1.0000x reference)
import functools

import jax
import jax.numpy as jnp
import numpy as np
from jax import lax
from jax.experimental import pallas as pl
from jax.experimental.pallas import tpu as pltpu
from jax.experimental.pallas import tpu_sc as plsc

B = 32
S = 4096
H = 768
K = 256
NC = 2
NS = 16
NV = S // 16
INT_MIN = np.int32(-2147483648)


def _sc_body(feat_hbm, logits_hbm, sel_hbm, idx_hbm, scr_hbm,
             logits_v, ukeys_v, aval_v, aidx_v, bval_v, bidx_v,
             gidx_v, rows_v, outsc_v, outix_v, sem, sem2):
    wid = lax.axis_index("s") * NC + lax.axis_index("c")
    iota16 = lax.iota(jnp.int32, 16)

    pltpu.sync_copy(logits_hbm.at[wid], logits_v)

    def p1(i, mvecs):
        new = []
        for u_ix in range(8):
            x = logits_v[pl.ds((i * 8 + u_ix) * 16, 16)]
            kb = lax.bitcast_convert_type(x, jnp.int32)
            u = kb ^ ((kb >> 31) | INT_MIN)
            ukeys_v[pl.ds((i * 8 + u_ix) * 16, 16)] = (
                lax.bitcast_convert_type(u, jnp.uint32))
            new.append(jnp.maximum(mvecs[u_ix], x))
        return tuple(new)
    minfs = tuple(jnp.full((16,), -jnp.inf, jnp.float32) for _ in range(8))
    mvecs = lax.fori_loop(0, NV // 8, p1, minfs)
    mvec = mvecs[0]
    for u_ix in range(1, 8):
        mvec = jnp.maximum(mvec, mvecs[u_ix])
    m = jnp.max(mvec)

    def p2(i, dvecs):
        new = []
        for u_ix in range(8):
            x = logits_v[pl.ds((i * 8 + u_ix) * 16, 16)]
            new.append(dvecs[u_ix] + jnp.exp(x - m))
        return tuple(new)
    zerosf = tuple(jnp.zeros((16,), jnp.float32) for _ in range(8))
    dvecs = lax.fori_loop(0, NV // 8, p2, zerosf)
    dvec = dvecs[0]
    for u_ix in range(1, 8):
        dvec = dvec + dvecs[u_ix]
    denom = jnp.sum(dvec)

    UNR = 16
    def bs(bi, t):
        cand = t | (jnp.uint32(1) << (jnp.uint32(31) - bi.astype(jnp.uint32)))
        def cnt_body(i, cs):
            base = i * (16 * UNR)
            new = []
            for u_ix in range(UNR):
                u = ukeys_v[pl.ds(base + u_ix * 16, 16)]
                new.append(cs[u_ix] + jnp.where(u >= cand, 1, 0).astype(jnp.int32))
            return tuple(new)
        zeros = tuple(jnp.zeros((16,), jnp.int32) for _ in range(UNR))
        cvecs = lax.fori_loop(0, NV // UNR, cnt_body, zeros)
        cvec = cvecs[0]
        for u_ix in range(1, UNR):
            cvec = cvec + cvecs[u_ix]
        cnt = jnp.sum(cvec)
        return jnp.where(cnt >= K, cand, t)
    t = lax.fori_loop(0, 32, bs, jnp.uint32(0))

    def comp_gt(i, off):
        u = ukeys_v[pl.ds(i * 16, 16)]
        mask = u > t
        ones = jnp.where(mask, 1, 0).astype(jnp.int32)
        pos = off + plsc.cumsum(ones) - 1
        plsc.store_scatter(aval_v, [pos], logits_v[pl.ds(i * 16, 16)], mask=mask)
        plsc.store_scatter(aidx_v, [pos], i * 16 + iota16, mask=mask)
        return off + jnp.sum(ones)
    cnt_gt = lax.fori_loop(0, NV, comp_gt, jnp.int32(0))

    def comp_eq(i, off):
        u = ukeys_v[pl.ds(i * 16, 16)]
        mask = u == t
        ones = jnp.where(mask, 1, 0).astype(jnp.int32)
        pos = off + plsc.cumsum(ones) - 1
        wmask = mask & (pos < K)
        plsc.store_scatter(aval_v, [pos], logits_v[pl.ds(i * 16, 16)], mask=wmask)
        plsc.store_scatter(aidx_v, [pos], i * 16 + iota16, mask=wmask)
        return off + jnp.sum(ones)
    lax.fori_loop(0, NV, comp_eq, cnt_gt)

    stages = []
    k = 2
    while k <= K:
        j = k // 2
        while j >= 1:
            stages.append((k, j))
            j //= 2
        k *= 2
    assert len(stages) % 2 == 0
    bufs = ((aval_v, aidx_v), (bval_v, bidx_v))
    for si, (k, j) in enumerate(stages):
        sval, sidx = bufs[si % 2]
        dval, didx = bufs[(si + 1) % 2]
        def stage(g, _, j=j, k=k, sval=sval, sidx=sidx, dval=dval, didx=didx):
            for sub in range(4):
                ivec = (g * 4 + sub) * 16 + iota16
                pvec = ivec ^ j
                av = sval[pl.ds((g * 4 + sub) * 16, 16)]
                ai = sidx[pl.ds((g * 4 + sub) * 16, 16)]
                bv = plsc.load_gather(sval, [pvec])
                bi = plsc.load_gather(sidx, [pvec])
                a_first = (av > bv) | ((av == bv) & (ai < bi))
                want_first = ((ivec & k) == 0) == ((ivec & j) == 0)
                cond = want_first == a_first
                dval[pl.ds((g * 4 + sub) * 16, 16)] = jnp.where(cond, av, bv)
                didx[pl.ds((g * 4 + sub) * 16, 16)] = jnp.where(cond, ai, bi)
            return 0
        lax.fori_loop(0, K // 64, stage, 0)

    base = wid * S
    for c in range(4):
        for g2 in range(4):
            gidx_v[c, pl.ds(g2 * 16, 16)] = (
                base + aidx_v[pl.ds(c * 64 + g2 * 16, 16)])
    sems = (sem, sem2)
    pend = [None, None]
    pend[0] = pltpu.async_copy(feat_hbm.at[gidx_v.at[0]], rows_v.at[0], sems[0])
    pend[1] = pltpu.async_copy(feat_hbm.at[gidx_v.at[1]], rows_v.at[1], sems[1])

    def outp(g, _):
        v = aval_v[pl.ds(g * 16, 16)]
        outsc_v[pl.ds(g * 16, 16)] = jnp.exp(v - m) / denom
        outix_v[pl.ds(g * 16, 16)] = aidx_v[pl.ds(g * 16, 16)]
        return 0
    lax.fori_loop(0, K // 16, outp, 0)
    pltpu.sync_copy(outsc_v, scr_hbm.at[wid])
    pltpu.sync_copy(outix_v, idx_hbm.at[wid])

    for c in range(4):
        pend[c % 2].wait()
        pltpu.sync_copy(rows_v.at[c % 2], sel_hbm.at[wid, pl.ds(c * 64, 64)])
        if c + 2 < 4:
            nb = c % 2
            pend[nb] = pltpu.async_copy(
                feat_hbm.at[gidx_v.at[c + 2]], rows_v.at[nb], sems[nb])


@functools.cache
def _build_select():
    return pl.kernel(
        _sc_body,
        out_type=(
            jax.ShapeDtypeStruct((B, K, H), jnp.float32),
            jax.ShapeDtypeStruct((B, K), jnp.int32),
            jax.ShapeDtypeStruct((B, K), jnp.float32),
        ),
        mesh=plsc.VectorSubcoreMesh(core_axis_name="c", subcore_axis_name="s",
                                    num_cores=NC, num_subcores=NS),
        scratch_types=[
            pltpu.VMEM((S,), jnp.float32),
            pltpu.VMEM((S,), jnp.uint32),
            pltpu.VMEM((K + 16,), jnp.float32),
            pltpu.VMEM((K + 16,), jnp.int32),
            pltpu.VMEM((K,), jnp.float32),
            pltpu.VMEM((K,), jnp.int32),
            pltpu.VMEM((4, 64), jnp.int32),
            pltpu.VMEM((2, 64, H), jnp.float32),
            pltpu.VMEM((K,), jnp.float32),
            pltpu.VMEM((K,), jnp.int32),
            pltpu.SemaphoreType.DMA,
            pltpu.SemaphoreType.DMA,
        ],
        compiler_params=pltpu.CompilerParams(needs_layout_passes=False),
    )


def kernel(vision_features, W, b, num_tokens):
    logits = jnp.squeeze(
        jnp.einsum('bsh,oh->bso', vision_features, W) + b, axis=-1)
    feat2d = vision_features.reshape(B * S, H)
    sel, idx, scores = _build_select()(feat2d, logits)
    idx = idx + (num_tokens - K)
    return (sel, idx, scores)
